# Initial kernel scaffold; baseline (speedup 1.0000x reference)
import jax
import jax.numpy as jnp
from jax.experimental import pallas as pl


def kernel(x, edge_index, edge_weight, W1, b1, W2, b2):
    raise NotImplementedError("write your pallas kernel here")


# speedup vs baseline: 83.8793x; 83.8793x over previous
import functools

import jax
import jax.numpy as jnp
from jax import lax
from jax.experimental import pallas as pl
from jax.experimental.pallas import tpu as pltpu
from jax.experimental.pallas import tpu_sc as plsc

NC = 2
NS = 16
NW = NC * NS
LANES = 16
ROW = 128
CHUNK_ROWS = 16
CHUNK = ROW * CHUNK_ROWS


def _mesh():
    return plsc.VectorSubcoreMesh(core_axis_name="c", subcore_axis_name="s",
                                  num_cores=NC, num_subcores=NS)


def _deg_kernel(n_nodes, rows_per_tile):
    @functools.partial(
        pl.kernel,
        out_type=jax.ShapeDtypeStruct((NW, n_nodes), jnp.float32),
        mesh=_mesh(),
        compiler_params=pltpu.CompilerParams(needs_layout_passes=False,
                                             use_tc_tiling_on_sc=False),
        scratch_types=[
            pltpu.VMEM((n_nodes,), jnp.float32),
            pltpu.VMEM((CHUNK_ROWS, ROW), jnp.int32),
            pltpu.VMEM((CHUNK_ROWS, ROW), jnp.float32),
        ],
    )
    def body(dst_hbm, ew_hbm, out_hbm, deg_v, dst_v, ew_v):
        c = lax.axis_index("c")
        s = lax.axis_index("s")
        w = s * NC + c
        zeros16 = jnp.zeros((LANES,), jnp.float32)

        def zstep(i, carry):
            deg_v[pl.ds(i * LANES, LANES)] = zeros16
            return carry

        lax.fori_loop(0, n_nodes // LANES, zstep, 0)

        row0 = w * rows_per_tile

        def chunk(k, carry):
            rbase = row0 + k * CHUNK_ROWS
            pltpu.sync_copy(dst_hbm.at[pl.ds(rbase, CHUNK_ROWS)], dst_v)
            pltpu.sync_copy(ew_hbm.at[pl.ds(rbase, CHUNK_ROWS)], ew_v)

            def row_loop(r, carry2):
                for t in range(ROW // LANES):
                    idx = dst_v[r, pl.ds(t * LANES, LANES)]
                    wv = ew_v[r, pl.ds(t * LANES, LANES)]
                    plsc.addupdate_scatter(deg_v, [idx], wv)
                return carry2

            lax.fori_loop(0, CHUNK_ROWS, row_loop, 0)
            return carry

        lax.fori_loop(0, rows_per_tile // CHUNK_ROWS, chunk, 0)
        pltpu.sync_copy(deg_v, out_hbm.at[w])

    return body


def _l1_kernel(n_nodes, rows_per_tile):
    nsl_a = ((n_nodes // NS) + 7) // 8 * 8
    nsl_last = n_nodes - (NS - 1) * nsl_a
    assert nsl_last > 0

    @functools.partial(
        pl.kernel,
        out_type=jax.ShapeDtypeStruct((NC, n_nodes, 8), jnp.float32),
        mesh=_mesh(),
        compiler_params=pltpu.CompilerParams(needs_layout_passes=False,
                                             use_tc_tiling_on_sc=False),
        scratch_types=[
            pltpu.VMEM((CHUNK_ROWS, ROW), jnp.int32),
            pltpu.VMEM((CHUNK_ROWS, ROW), jnp.int32),
            pltpu.VMEM((CHUNK_ROWS, ROW), jnp.float32),
            pltpu.VMEM((CHUNK_ROWS, ROW), jnp.float32),
            pltpu.VMEM((CHUNK, 8), jnp.float32),
            pltpu.VMEM_SHARED((n_nodes,), jnp.float32),
            pltpu.VMEM_SHARED((n_nodes, 8), jnp.float32),
            pltpu.SemaphoreType.DMA,
            pltpu.SemaphoreType.DMA,
        ],
    )
    def body(src_hbm, dst_hbm, ew_hbm, dis_hbm, x_hbm, z_hbm, out_hbm,
             src_v, dst_v, ew_v, se_v, xg, dis_sh, acc_sh, sem, sem2):
        c = lax.axis_index("c")
        s = lax.axis_index("s")
        w = s * NC + c

        @pl.when(s == 0)
        def _():
            pltpu.sync_copy(dis_hbm, dis_sh)

        @pl.when(s < NS - 1)
        def _():
            pltpu.sync_copy(z_hbm.at[pl.ds(s * nsl_a, nsl_a), :],
                            acc_sh.at[pl.ds(s * nsl_a, nsl_a), :])

        @pl.when(s == NS - 1)
        def _():
            off = (NS - 1) * nsl_a
            pltpu.sync_copy(z_hbm.at[pl.ds(off, nsl_last), :],
                            acc_sh.at[pl.ds(off, nsl_last), :])

        plsc.subcore_barrier()

        iota16 = lax.iota(jnp.int32, LANES)
        cols = [jnp.full((LANES,), col, jnp.int32) for col in range(8)]
        row0 = w * rows_per_tile

        def chunk(k, carry):
            rbase = row0 + k * CHUNK_ROWS
            pltpu.sync_copy(src_hbm.at[pl.ds(rbase, CHUNK_ROWS)], src_v)
            pltpu.sync_copy(dst_hbm.at[pl.ds(rbase, CHUNK_ROWS)], dst_v)
            pltpu.sync_copy(ew_hbm.at[pl.ds(rbase, CHUNK_ROWS)], ew_v)
            descs = []
            for j in range(CHUNK_ROWS):
                descs.append(pltpu.async_copy(
                    x_hbm.at[src_v.at[j]],
                    xg.at[pl.ds(j * ROW, ROW), :], sem))
                descs.append(pltpu.async_copy(
                    dis_sh.at[src_v.at[j]], se_v.at[j], sem2))
            for d in descs:
                d.wait()

            def se_row(r, carry2):
                for t in range(ROW // LANES):
                    d16 = se_v[r, pl.ds(t * LANES, LANES)]
                    w16 = ew_v[r, pl.ds(t * LANES, LANES)]
                    se_v[r, pl.ds(t * LANES, LANES)] = d16 * w16
                return carry2

            lax.fori_loop(0, CHUNK_ROWS, se_row, 0)

            def blk(j, carry2):
                def grp(g, carry3):
                    rowi = j * ROW + g * LANES + iota16
                    se16 = se_v[j, pl.ds(g * LANES, LANES)]
                    for col in range(8):
                        v = plsc.load_gather(xg, [rowi, cols[col]])
                        plsc.store_scatter(xg, [rowi, cols[col]], v * se16)
                    return carry3

                lax.fori_loop(0, ROW // LANES, grp, 0)
                return carry2

            lax.fori_loop(0, CHUNK_ROWS, blk, 0)
            for j in range(CHUNK_ROWS):
                pltpu.sync_copy(xg.at[pl.ds(j * ROW, ROW), :],
                                acc_sh.at[dst_v.at[j]], add=True)
            return carry

        lax.fori_loop(0, rows_per_tile // CHUNK_ROWS, chunk, 0)
        plsc.subcore_barrier()

        @pl.when(s < NS - 1)
        def _():
            pltpu.sync_copy(acc_sh.at[pl.ds(s * nsl_a, nsl_a), :],
                            out_hbm.at[c, pl.ds(s * nsl_a, nsl_a), :])

        @pl.when(s == NS - 1)
        def _():
            off = (NS - 1) * nsl_a
            pltpu.sync_copy(acc_sh.at[pl.ds(off, nsl_last), :],
                            out_hbm.at[c, pl.ds(off, nsl_last), :])

    return body


def _l2_kernel(n_nodes, rows_per_tile):
    nsl_a = ((n_nodes // NS) + 7) // 8 * 8
    nsl_last = n_nodes - (NS - 1) * nsl_a
    assert nsl_last > 0

    @functools.partial(
        pl.kernel,
        out_type=jax.ShapeDtypeStruct((NC, n_nodes), jnp.float32),
        mesh=_mesh(),
        compiler_params=pltpu.CompilerParams(needs_layout_passes=False,
                                             use_tc_tiling_on_sc=False),
        scratch_types=[
            pltpu.VMEM((n_nodes,), jnp.float32),
            pltpu.VMEM((CHUNK_ROWS, ROW), jnp.int32),
            pltpu.VMEM((CHUNK_ROWS, ROW), jnp.int32),
            pltpu.VMEM((CHUNK_ROWS, ROW), jnp.float32),
            pltpu.VMEM((CHUNK_ROWS, ROW), jnp.float32),
            pltpu.VMEM_SHARED((n_nodes,), jnp.float32),
        ],
    )
    def body(src_hbm, dst_hbm, ew_hbm, hws_hbm, z_hbm, out_hbm,
             hws_v, src_v, dst_v, ew_v, val_v, acc_sh):
        c = lax.axis_index("c")
        s = lax.axis_index("s")
        w = s * NC + c
        pltpu.sync_copy(hws_hbm, hws_v)

        @pl.when(s < NS - 1)
        def _():
            pltpu.sync_copy(z_hbm.at[pl.ds(s * nsl_a, nsl_a)],
                            acc_sh.at[pl.ds(s * nsl_a, nsl_a)])

        @pl.when(s == NS - 1)
        def _():
            off = (NS - 1) * nsl_a
            pltpu.sync_copy(z_hbm.at[pl.ds(off, nsl_last)],
                            acc_sh.at[pl.ds(off, nsl_last)])

        plsc.subcore_barrier()
        row0 = w * rows_per_tile

        def chunk(k, carry):
            rbase = row0 + k * CHUNK_ROWS
            pltpu.sync_copy(src_hbm.at[pl.ds(rbase, CHUNK_ROWS)], src_v)
            pltpu.sync_copy(dst_hbm.at[pl.ds(rbase, CHUNK_ROWS)], dst_v)
            pltpu.sync_copy(ew_hbm.at[pl.ds(rbase, CHUNK_ROWS)], ew_v)

            def row_loop(r, carry2):
                for t in range(ROW // LANES):
                    s16 = src_v[r, pl.ds(t * LANES, LANES)]
                    w16 = ew_v[r, pl.ds(t * LANES, LANES)]
                    h16 = plsc.load_gather(hws_v, [s16])
                    val_v[r, pl.ds(t * LANES, LANES)] = h16 * w16
                return carry2

            lax.fori_loop(0, CHUNK_ROWS, row_loop, 0)
            for j in range(CHUNK_ROWS):
                pltpu.sync_copy(val_v.at[j], acc_sh.at[dst_v.at[j]], add=True)
            return carry

        lax.fori_loop(0, rows_per_tile // CHUNK_ROWS, chunk, 0)
        plsc.subcore_barrier()

        @pl.when(s < NS - 1)
        def _():
            pltpu.sync_copy(acc_sh.at[pl.ds(s * nsl_a, nsl_a)],
                            out_hbm.at[c, pl.ds(s * nsl_a, nsl_a)])

        @pl.when(s == NS - 1)
        def _():
            off = (NS - 1) * nsl_a
            pltpu.sync_copy(acc_sh.at[pl.ds(off, nsl_last)],
                            out_hbm.at[c, pl.ds(off, nsl_last)])

    return body


def _dis_body(degp_ref, dis_ref):
    deg = jnp.sum(degp_ref[...], axis=0, keepdims=True) + 1.0
    dis_ref[...] = lax.rsqrt(deg)


def _dense_body(accp_ref, dis_ref, x_ref, w1_ref, b1_ref, w2t_ref, out_ref):
    dis = dis_ref[...]
    a = (accp_ref[0] + accp_ref[1]) * dis + x_ref[...] * (dis * dis)
    h = jnp.dot(a, w1_ref[...], preferred_element_type=jnp.float32)
    h = jnp.maximum(h + b1_ref[...], 0.0)
    hw = jnp.sum(h * w2t_ref[...], axis=1, keepdims=True)
    out_ref[...] = hw * dis


def _fin_body(accp_ref, hws_ref, dis_ref, b2_ref, out_ref):
    acc = jnp.sum(accp_ref[...], axis=0, keepdims=True) + hws_ref[...]
    out_ref[...] = acc * dis_ref[...] + b2_ref[...]


def kernel(x, edge_index, edge_weight, W1, b1, W2, b2):
    n = x.shape[0]
    e = edge_weight.shape[0]
    assert n % (NS * LANES // 16) == 0 and n % LANES == 0, n

    src = edge_index[0].astype(jnp.int32)
    dst = edge_index[1].astype(jnp.int32)
    ew = edge_weight.astype(jnp.float32)

    grain = NW * CHUNK
    ep = ((e + grain - 1) // grain) * grain
    pad = ep - e
    if pad:
        padidx = jnp.arange(pad, dtype=jnp.int32) % n
        src = jnp.concatenate([src, padidx])
        dst = jnp.concatenate([dst, padidx])
        ew = jnp.concatenate([ew, jnp.zeros((pad,), jnp.float32)])
    rows = ep // ROW
    rows_per_tile = rows // NW
    src2 = src.reshape(rows, ROW)
    dst2 = dst.reshape(rows, ROW)
    ew2 = ew.reshape(rows, ROW)

    zeros8 = jnp.zeros((n, 8), jnp.float32)
    zeros1 = jnp.zeros((n,), jnp.float32)

    degp = _deg_kernel(n, rows_per_tile)(dst2, ew2)

    dis = pl.pallas_call(
        _dis_body,
        out_shape=jax.ShapeDtypeStruct((1, n), jnp.float32),
    )(degp)
    dis_flat = dis.reshape(n)
    dis_col = dis.reshape(n, 1)

    acc1p = _l1_kernel(n, rows_per_tile)(src2, dst2, ew2, dis_flat, x, zeros8)

    bn = 4000
    hws = pl.pallas_call(
        _dense_body,
        grid=(n // bn,),
        in_specs=[
            pl.BlockSpec((NC, bn, 8), lambda i: (0, i, 0)),
            pl.BlockSpec((bn, 1), lambda i: (i, 0)),
            pl.BlockSpec((bn, 8), lambda i: (i, 0)),
            pl.BlockSpec((8, 64), lambda i: (0, 0)),
            pl.BlockSpec((1, 64), lambda i: (0, 0)),
            pl.BlockSpec((1, 64), lambda i: (0, 0)),
        ],
        out_specs=pl.BlockSpec((bn, 1), lambda i: (i, 0)),
        out_shape=jax.ShapeDtypeStruct((n, 1), jnp.float32),
    )(acc1p, dis_col, x, W1, b1.reshape(1, 64), W2.reshape(1, 64))

    acc2p = _l2_kernel(n, rows_per_tile)(src2, dst2, ew2, hws.reshape(n),
                                         zeros1)

    out = pl.pallas_call(
        _fin_body,
        out_shape=jax.ShapeDtypeStruct((1, n), jnp.float32),
    )(acc2p, hws.reshape(1, n), dis, b2.reshape(1, 1))
    return out.reshape(n, 1)


# speedup vs baseline: 109.8668x; 1.3098x over previous
import functools

import jax
import jax.numpy as jnp
from jax import lax
from jax.experimental import pallas as pl
from jax.experimental.pallas import tpu as pltpu
from jax.experimental.pallas import tpu_sc as plsc

NC = 2
NS = 16
NW = NC * NS
LANES = 16
ROW = 128
CHUNK_ROWS = 16
CHUNK = ROW * CHUNK_ROWS


def _mesh():
    return plsc.VectorSubcoreMesh(core_axis_name="c", subcore_axis_name="s",
                                  num_cores=NC, num_subcores=NS)


_SC_PARAMS = pltpu.CompilerParams(needs_layout_passes=False,
                                  use_tc_tiling_on_sc=False)


def _deg_kernel(n_nodes, rows_per_tile):
    nchunks = rows_per_tile // CHUNK_ROWS
    assert nchunks % 2 == 0

    @functools.partial(
        pl.kernel,
        out_type=jax.ShapeDtypeStruct((NW, n_nodes), jnp.float32),
        mesh=_mesh(),
        compiler_params=_SC_PARAMS,
        scratch_types=[
            pltpu.VMEM((n_nodes,), jnp.float32),
            pltpu.VMEM((2, CHUNK_ROWS, ROW), jnp.int32),
            pltpu.VMEM((2, CHUNK_ROWS, ROW), jnp.float32),
            pltpu.SemaphoreType.DMA,
        ],
    )
    def body(dst_hbm, ew_hbm, out_hbm, deg_v, dst_v, ew_v, isem):
        c = lax.axis_index("c")
        s = lax.axis_index("s")
        w = s * NC + c
        zeros16 = jnp.zeros((LANES,), jnp.float32)

        def zstep(i, carry):
            deg_v[pl.ds(i * LANES, LANES)] = zeros16
            return carry

        lax.fori_loop(0, n_nodes // LANES, zstep, 0)

        row0 = w * rows_per_tile

        def fetch(k, b):
            rbase = row0 + k * CHUNK_ROWS
            pltpu.async_copy(dst_hbm.at[pl.ds(rbase, CHUNK_ROWS)],
                             dst_v.at[b], isem)
            pltpu.async_copy(ew_hbm.at[pl.ds(rbase, CHUNK_ROWS)],
                             ew_v.at[b], isem)

        def wait_idx(b):
            pltpu.make_async_copy(dst_hbm.at[pl.ds(0, CHUNK_ROWS)],
                                  dst_v.at[b], isem).wait()
            pltpu.make_async_copy(ew_hbm.at[pl.ds(0, CHUNK_ROWS)],
                                  ew_v.at[b], isem).wait()

        fetch(0, 0)

        def pair(kk, carry):
            for b in range(2):
                k = kk * 2 + b
                wait_idx(b)

                @pl.when(k + 1 < nchunks)
                def _():
                    fetch(k + 1, 1 - b)

                def row_loop(r, carry2):
                    for t in range(ROW // LANES):
                        idx = dst_v[b, r, pl.ds(t * LANES, LANES)]
                        wv = ew_v[b, r, pl.ds(t * LANES, LANES)]
                        plsc.addupdate_scatter(deg_v, [idx], wv)
                    return carry2

                lax.fori_loop(0, CHUNK_ROWS, row_loop, 0)
            return carry

        lax.fori_loop(0, nchunks // 2, pair, 0)
        pltpu.sync_copy(deg_v, out_hbm.at[w])

    return body


def _l1_kernel(n_nodes, rows_per_tile):
    nchunks = rows_per_tile // CHUNK_ROWS
    assert nchunks % 2 == 0
    nsl_a = ((n_nodes // NS) + 7) // 8 * 8
    nsl_last = n_nodes - (NS - 1) * nsl_a
    assert nsl_last > 0

    @functools.partial(
        pl.kernel,
        out_type=jax.ShapeDtypeStruct((NC, n_nodes, 8), jnp.float32),
        mesh=_mesh(),
        compiler_params=_SC_PARAMS,
        scratch_types=[
            pltpu.VMEM((2, CHUNK_ROWS, ROW), jnp.int32),
            pltpu.VMEM((2, CHUNK_ROWS, ROW), jnp.int32),
            pltpu.VMEM((2, CHUNK_ROWS, ROW), jnp.float32),
            pltpu.VMEM((2, CHUNK, 8), jnp.float32),
            pltpu.VMEM_SHARED((n_nodes, 8), jnp.float32),
            pltpu.SemaphoreType.DMA,
            pltpu.SemaphoreType.DMA,
            pltpu.SemaphoreType.DMA,
        ],
    )
    def body(src_hbm, dst_hbm, ew_hbm, xs_hbm, z_hbm, out_hbm,
             src_v, dst_v, ew_v, xg, acc_sh, isem, gsem, ssem):
        c = lax.axis_index("c")
        s = lax.axis_index("s")
        w = s * NC + c

        @pl.when(s < NS - 1)
        def _():
            pltpu.sync_copy(z_hbm.at[pl.ds(s * nsl_a, nsl_a), :],
                            acc_sh.at[pl.ds(s * nsl_a, nsl_a), :])

        @pl.when(s == NS - 1)
        def _():
            off = (NS - 1) * nsl_a
            pltpu.sync_copy(z_hbm.at[pl.ds(off, nsl_last), :],
                            acc_sh.at[pl.ds(off, nsl_last), :])

        plsc.subcore_barrier()

        iota16 = lax.iota(jnp.int32, LANES)
        cols = [jnp.full((LANES,), col, jnp.int32) for col in range(8)]
        row0 = w * rows_per_tile

        def fetch(k, b):
            rbase = row0 + k * CHUNK_ROWS
            pltpu.async_copy(src_hbm.at[pl.ds(rbase, CHUNK_ROWS)],
                             src_v.at[b], isem)
            pltpu.async_copy(dst_hbm.at[pl.ds(rbase, CHUNK_ROWS)],
                             dst_v.at[b], isem)
            pltpu.async_copy(ew_hbm.at[pl.ds(rbase, CHUNK_ROWS)],
                             ew_v.at[b], isem)

        def wait_idx(b):
            pltpu.make_async_copy(src_hbm.at[pl.ds(0, CHUNK_ROWS)],
                                  src_v.at[b], isem).wait()
            pltpu.make_async_copy(dst_hbm.at[pl.ds(0, CHUNK_ROWS)],
                                  dst_v.at[b], isem).wait()
            pltpu.make_async_copy(ew_hbm.at[pl.ds(0, CHUNK_ROWS)],
                                  ew_v.at[b], isem).wait()

        def fire_gathers(b):
            for j in range(CHUNK_ROWS):
                pltpu.async_copy(xs_hbm.at[src_v.at[b, j]],
                                 xg.at[b, pl.ds(j * ROW, ROW), :], gsem)

        def wait_xg(b):
            pltpu.make_async_copy(xs_hbm.at[pl.ds(0, CHUNK)],
                                  xg.at[b], gsem).wait()

        def drain_scatter(b):
            pltpu.make_async_copy(xs_hbm.at[pl.ds(0, CHUNK)],
                                  xg.at[b], ssem).wait()

        fetch(0, 0)

        def pair(kk, carry):
            for b in range(2):
                k = kk * 2 + b
                wait_idx(b)
                fire_gathers(b)

                @pl.when(k + 1 < nchunks)
                def _():
                    fetch(k + 1, 1 - b)

                wait_xg(b)

                def blk(j, carry2):
                    def grp(g, carry3):
                        rowi = j * ROW + g * LANES + iota16
                        se16 = ew_v[b, j, pl.ds(g * LANES, LANES)]
                        xgb = xg.at[b]
                        for col in range(8):
                            v = plsc.load_gather(xgb, [rowi, cols[col]])
                            plsc.store_scatter(xgb, [rowi, cols[col]],
                                               v * se16)
                        return carry3

                    lax.fori_loop(0, ROW // LANES, grp, 0)
                    return carry2

                lax.fori_loop(0, CHUNK_ROWS, blk, 0)

                for j in range(CHUNK_ROWS):
                    pltpu.async_copy(xg.at[b, pl.ds(j * ROW, ROW), :],
                                     acc_sh.at[dst_v.at[b, j]], ssem,
                                     add=True)

                drain_scatter(b)
            return carry

        lax.fori_loop(0, nchunks // 2, pair, 0)
        plsc.subcore_barrier()

        @pl.when(s < NS - 1)
        def _():
            pltpu.sync_copy(acc_sh.at[pl.ds(s * nsl_a, nsl_a), :],
                            out_hbm.at[c, pl.ds(s * nsl_a, nsl_a), :])

        @pl.when(s == NS - 1)
        def _():
            off = (NS - 1) * nsl_a
            pltpu.sync_copy(acc_sh.at[pl.ds(off, nsl_last), :],
                            out_hbm.at[c, pl.ds(off, nsl_last), :])

    return body


def _l2_kernel(n_nodes, rows_per_tile):
    nchunks = rows_per_tile // CHUNK_ROWS
    assert nchunks % 2 == 0
    nsl_a = ((n_nodes // NS) + 7) // 8 * 8
    nsl_last = n_nodes - (NS - 1) * nsl_a
    assert nsl_last > 0

    @functools.partial(
        pl.kernel,
        out_type=jax.ShapeDtypeStruct((NW, n_nodes), jnp.float32),
        mesh=_mesh(),
        compiler_params=_SC_PARAMS,
        scratch_types=[
            pltpu.VMEM((n_nodes,), jnp.float32),
            pltpu.VMEM((2, CHUNK_ROWS, ROW), jnp.int32),
            pltpu.VMEM((2, CHUNK_ROWS, ROW), jnp.int32),
            pltpu.VMEM((2, CHUNK_ROWS, ROW), jnp.float32),
            pltpu.VMEM((2, CHUNK), jnp.float32),
            pltpu.VMEM_SHARED((n_nodes,), jnp.float32),
            pltpu.SemaphoreType.DMA,
            pltpu.SemaphoreType.DMA,
        ],
    )
    def body(src_hbm, dst_hbm, ew_hbm, hws_hbm, out_hbm,
             acc_v, src_v, dst_v, ew_v, hg_v, hws_sh, isem, gsem):
        c = lax.axis_index("c")
        s = lax.axis_index("s")
        w = s * NC + c

        @pl.when(s < NS - 1)
        def _():
            pltpu.sync_copy(hws_hbm.at[pl.ds(s * nsl_a, nsl_a)],
                            hws_sh.at[pl.ds(s * nsl_a, nsl_a)])

        @pl.when(s == NS - 1)
        def _():
            off = (NS - 1) * nsl_a
            pltpu.sync_copy(hws_hbm.at[pl.ds(off, nsl_last)],
                            hws_sh.at[pl.ds(off, nsl_last)])

        zeros16 = jnp.zeros((LANES,), jnp.float32)

        def zstep(i, carry):
            acc_v[pl.ds(i * LANES, LANES)] = zeros16
            return carry

        lax.fori_loop(0, n_nodes // LANES, zstep, 0)
        plsc.subcore_barrier()
        row0 = w * rows_per_tile

        def fetch(k, b):
            rbase = row0 + k * CHUNK_ROWS
            pltpu.async_copy(src_hbm.at[pl.ds(rbase, CHUNK_ROWS)],
                             src_v.at[b], isem)
            pltpu.async_copy(dst_hbm.at[pl.ds(rbase, CHUNK_ROWS)],
                             dst_v.at[b], isem)
            pltpu.async_copy(ew_hbm.at[pl.ds(rbase, CHUNK_ROWS)],
                             ew_v.at[b], isem)

        def wait_idx(b):
            pltpu.make_async_copy(src_hbm.at[pl.ds(0, CHUNK_ROWS)],
                                  src_v.at[b], isem).wait()
            pltpu.make_async_copy(dst_hbm.at[pl.ds(0, CHUNK_ROWS)],
                                  dst_v.at[b], isem).wait()
            pltpu.make_async_copy(ew_hbm.at[pl.ds(0, CHUNK_ROWS)],
                                  ew_v.at[b], isem).wait()

        def fire_gathers(b):
            for j in range(CHUNK_ROWS):
                pltpu.async_copy(hws_sh.at[src_v.at[b, j]],
                                 hg_v.at[b, pl.ds(j * ROW, ROW)], gsem)

        def wait_hg(b):
            pltpu.make_async_copy(hws_sh.at[pl.ds(0, CHUNK)],
                                  hg_v.at[b], gsem).wait()

        fetch(0, 0)

        def pair(kk, carry):
            for b in range(2):
                k = kk * 2 + b
                wait_idx(b)
                fire_gathers(b)

                @pl.when(k + 1 < nchunks)
                def _():
                    fetch(k + 1, 1 - b)

                wait_hg(b)

                def row_loop(r, carry2):
                    for t in range(ROW // LANES):
                        d16 = dst_v[b, r, pl.ds(t * LANES, LANES)]
                        w16 = ew_v[b, r, pl.ds(t * LANES, LANES)]
                        h16 = hg_v[b, pl.ds(r * ROW + t * LANES, LANES)]
                        plsc.addupdate_scatter(acc_v, [d16], h16 * w16)
                    return carry2

                lax.fori_loop(0, CHUNK_ROWS, row_loop, 0)
            return carry

        lax.fori_loop(0, nchunks // 2, pair, 0)
        pltpu.sync_copy(acc_v, out_hbm.at[w])

    return body


def _dis_body(degp_ref, dis_ref):
    deg = jnp.sum(degp_ref[...], axis=0, keepdims=True) + 1.0
    dis_ref[...] = lax.rsqrt(deg)


def _xs_body(x_ref, dis_ref, xs_ref):
    xs_ref[...] = x_ref[...] * dis_ref[...]


def _dense_body(accp_ref, dis_ref, x_ref, w1_ref, b1_ref, w2t_ref, out_ref):
    dis = dis_ref[...]
    a = (accp_ref[0] + accp_ref[1]) * dis + x_ref[...] * (dis * dis)
    h = jnp.dot(a, w1_ref[...], preferred_element_type=jnp.float32)
    h = jnp.maximum(h + b1_ref[...], 0.0)
    hw = jnp.sum(h * w2t_ref[...], axis=1, keepdims=True)
    out_ref[...] = hw * dis


def _fin_body(accp_ref, hws_ref, dis_ref, b2_ref, out_ref):
    acc = jnp.sum(accp_ref[...], axis=0, keepdims=True) + hws_ref[...]
    out_ref[...] = acc * dis_ref[...] + b2_ref[...]


def kernel(x, edge_index, edge_weight, W1, b1, W2, b2):
    n = x.shape[0]
    e = edge_weight.shape[0]
    assert n % LANES == 0, n

    src = edge_index[0].astype(jnp.int32)
    dst = edge_index[1].astype(jnp.int32)
    ew = edge_weight.astype(jnp.float32)

    grain = NW * CHUNK * 2
    ep = ((e + grain - 1) // grain) * grain
    pad = ep - e
    if pad:
        padidx = jnp.arange(pad, dtype=jnp.int32) % n
        src = jnp.concatenate([src, padidx])
        dst = jnp.concatenate([dst, padidx])
        ew = jnp.concatenate([ew, jnp.zeros((pad,), jnp.float32)])
    rows = ep // ROW
    rows_per_tile = rows // NW
    src2 = src.reshape(rows, ROW)
    dst2 = dst.reshape(rows, ROW)
    ew2 = ew.reshape(rows, ROW)

    zeros8 = jnp.zeros((n, 8), jnp.float32)

    degp = _deg_kernel(n, rows_per_tile)(dst2, ew2)

    dis = pl.pallas_call(
        _dis_body,
        out_shape=jax.ShapeDtypeStruct((1, n), jnp.float32),
    )(degp)
    dis_col = dis.reshape(n, 1)
    bn = 4000
    xs = pl.pallas_call(
        _xs_body,
        grid=(n // bn,),
        in_specs=[
            pl.BlockSpec((bn, 8), lambda i: (i, 0)),
            pl.BlockSpec((bn, 1), lambda i: (i, 0)),
        ],
        out_specs=pl.BlockSpec((bn, 8), lambda i: (i, 0)),
        out_shape=jax.ShapeDtypeStruct((n, 8), jnp.float32),
    )(x, dis_col)

    acc1p = _l1_kernel(n, rows_per_tile)(src2, dst2, ew2, xs, zeros8)

    hws = pl.pallas_call(
        _dense_body,
        grid=(n // bn,),
        in_specs=[
            pl.BlockSpec((NC, bn, 8), lambda i: (0, i, 0)),
            pl.BlockSpec((bn, 1), lambda i: (i, 0)),
            pl.BlockSpec((bn, 8), lambda i: (i, 0)),
            pl.BlockSpec((8, 64), lambda i: (0, 0)),
            pl.BlockSpec((1, 64), lambda i: (0, 0)),
            pl.BlockSpec((1, 64), lambda i: (0, 0)),
        ],
        out_specs=pl.BlockSpec((bn, 1), lambda i: (i, 0)),
        out_shape=jax.ShapeDtypeStruct((n, 1), jnp.float32),
    )(acc1p, dis_col, x, W1, b1.reshape(1, 64), W2.reshape(1, 64))

    acc2p = _l2_kernel(n, rows_per_tile)(src2, dst2, ew2, hws.reshape(n))

    out = pl.pallas_call(
        _fin_body,
        out_shape=jax.ShapeDtypeStruct((1, n), jnp.float32),
    )(acc2p, hws.reshape(1, n), dis, b2.reshape(1, 1))
    return out.reshape(n, 1)
